# baseline (speedup 1.0000x reference)
import jax
import jax.numpy as jnp
from jax.experimental import pallas as pl
from jax.experimental.pallas import tpu as pltpu

_N = 8
_S = 4
_H = 8
_ETA = 1.0

_DN_T = (((1,), (1,)), ((), ()))
_DN_N = (((1,), (0,)), ((), ()))
_NK = _N * (_H + _S + 1)


def _glo_kernel(x_ref, w_ref, b_ref, wcat_ref, mw_ref, m2_ref, slope_ref,
                center_ref, gcatt_ref, out_ref, wall_s, bw_s):
    f32 = jnp.float32
    bf16 = jnp.bfloat16
    D = w_ref.shape[0]

    @pl.when(pl.program_id(0) == 0)
    def _prep():
        w = w_ref[...]
        wall_s[0:D, :] = w.astype(bf16)
        wall_s[D:, :] = jax.lax.dot_general(
            wcat_ref[...], w, (((0,), (0,)), ((), ())),
            preferred_element_type=f32).astype(bf16)
        bw_s[...] = jax.lax.dot_general(
            b_ref[...], wcat_ref[...], _DN_N, preferred_element_type=f32)

    x_bf = x_ref[...].astype(bf16)
    y = jax.lax.dot_general(
        x_bf, wall_s[...], _DN_T, preferred_element_type=f32)
    x_lin = y[:, :D] + b_ref[...]
    aux = y[:, D:] + bw_s[...]
    u_all = aux[:, :_N * _H]
    proj = aux[:, _N * _H:_N * _H + _N * _S] - mw_ref[...]
    xm = aux[:, _N * _H + _N * _S:]

    r2 = jnp.sum(x_lin * x_lin, axis=1, keepdims=True)
    d2 = r2 - 2.0 * xm + m2_ref[...]
    q = proj * proj
    smat = (jax.lax.broadcasted_iota(jnp.int32, (_N * _S, _N), 0) // _S ==
            jax.lax.broadcasted_iota(jnp.int32, (_N * _S, _N), 1)).astype(f32)
    qsum = jax.lax.dot_general(
        q, smat, _DN_N, preferred_element_type=f32)
    score = slope_ref[...] * (qsum / d2 - center_ref[...])

    rowmax = jnp.max(score, axis=1, keepdims=True)
    tb = x_bf.shape[0]
    iota_n = jax.lax.broadcasted_iota(jnp.int32, (tb, _N), 1)
    idx = jnp.min(jnp.where(score == rowmax, iota_n, _N),
                  axis=1, keepdims=True)
    ss = jax.nn.sigmoid(rowmax)

    nh = _N * _H
    vals = jnp.concatenate(
        [u_all, jnp.ones((tb, _N), dtype=f32)], axis=1)
    lbl = jax.lax.broadcasted_iota(jnp.int32, (tb, nh + _N), 1)
    lbl = jnp.where(lbl < nh, lbl // _H, lbl - nh)
    masked = (jnp.where(lbl == idx, vals, 0.0) * ss).astype(bf16)
    upd_ss = jax.lax.dot_general(
        masked, gcatt_ref[...].astype(bf16), _DN_T,
        preferred_element_type=f32)

    out_ref[...] = (1.0 - ss) * x_lin + _ETA * upd_ss


def kernel(x, W_lin, b_lin, select_weight, select_mean_diff, imp_slope,
           imp_center, lora_update, lora_degen, bias_p, debias_p):
    B, T, D = x.shape
    N, _, S = select_weight.shape
    H = lora_update.shape[2]
    x2 = x.reshape(B * T, D)
    b2 = b_lin.reshape(1, D)
    slope = imp_slope.reshape(1, N)
    center = imp_center.reshape(1, N)

    wsel = jnp.transpose(select_weight, (1, 0, 2)).reshape(D, N * S)
    u2 = jnp.transpose(lora_update, (1, 0, 2)).reshape(D, N * H)
    wcat = jnp.concatenate([u2, wsel, select_mean_diff.T], axis=1)
    mw = jnp.einsum('nd,nds->ns', select_mean_diff,
                    select_weight).reshape(1, N * S)
    m2 = jnp.sum(select_mean_diff * select_mean_diff, axis=1).reshape(1, N)
    c = jnp.einsum('nd,ndh->nh', debias_p, lora_update)
    cb = jnp.einsum('nh,ndh->nd', c, lora_degen)
    g2t = jnp.transpose(lora_degen, (1, 0, 2)).reshape(D, N * H)
    gcatt = jnp.concatenate([g2t, (bias_p - cb).T], axis=1)

    TB = 512
    grid = ((B * T) // TB,)
    const = lambda shape: pl.BlockSpec(shape, lambda i: (0, 0))
    out = pl.pallas_call(
        _glo_kernel,
        grid=grid,
        in_specs=[
            pl.BlockSpec((TB, D), lambda i: (i, 0)),
            const((D, D)),
            const((1, D)),
            const((D, _NK)),
            const((1, N * S)),
            const((1, N)),
            const((1, N)),
            const((1, N)),
            const((D, N * (H + 1))),
        ],
        out_specs=pl.BlockSpec((TB, D), lambda i: (i, 0)),
        out_shape=jax.ShapeDtypeStruct((B * T, D), jnp.float32),
        scratch_shapes=[
            pltpu.VMEM((D + _NK, D), jnp.bfloat16),
            pltpu.VMEM((1, _NK), jnp.float32),
        ],
        compiler_params=pltpu.CompilerParams(
            dimension_semantics=("arbitrary",)),
    )(x2, W_lin, b2, wcat, mw, m2, slope, center, gcatt)
    return out.reshape(B, T, D)


# speedup vs baseline: 1.0403x; 1.0403x over previous
import jax
import jax.numpy as jnp
from jax.experimental import pallas as pl
from jax.experimental.pallas import tpu as pltpu

_N = 8
_S = 4
_H = 8
_ETA = 1.0

_DN_T = (((1,), (1,)), ((), ()))
_DN_N = (((1,), (0,)), ((), ()))
_NK = _N * (_H + _S + 1)


def _glo_kernel(x_ref, w_ref, b_ref, wcat_ref, mw_ref, m2_ref, slope_ref,
                center_ref, gcat_ref, out_ref, wall_s, bw_s):
    f32 = jnp.float32
    bf16 = jnp.bfloat16
    D = w_ref.shape[0]

    @pl.when(pl.program_id(0) == 0)
    def _prep():
        w = w_ref[...]
        wall_s[0:D, :] = w.astype(bf16)
        wall_s[D:, :] = jax.lax.dot_general(
            wcat_ref[...], w, (((0,), (0,)), ((), ())),
            preferred_element_type=f32).astype(bf16)
        bw_s[...] = jax.lax.dot_general(
            b_ref[...], wcat_ref[...], _DN_N, preferred_element_type=f32)

    x_bf = x_ref[...].astype(bf16)
    y = jax.lax.dot_general(
        x_bf, wall_s[...], _DN_T, preferred_element_type=f32)
    x_lin = y[:, :D] + b_ref[...]
    aux = y[:, D:] + bw_s[...]
    u_all = aux[:, :_N * _H]
    proj = aux[:, _N * _H:_N * _H + _N * _S] - mw_ref[...]
    xm = aux[:, _N * _H + _N * _S:]

    r2 = jnp.sum(x_lin * x_lin, axis=1, keepdims=True)
    d2 = r2 - 2.0 * xm + m2_ref[...]
    q = proj * proj
    smat = (jax.lax.broadcasted_iota(jnp.int32, (_N * _S, _N), 0) // _S ==
            jax.lax.broadcasted_iota(jnp.int32, (_N * _S, _N), 1)).astype(f32)
    qsum = jax.lax.dot_general(
        q, smat, _DN_N, preferred_element_type=f32)
    score = slope_ref[...] * (qsum / d2 - center_ref[...])

    rowmax = jnp.max(score, axis=1, keepdims=True)
    tb = x_bf.shape[0]
    iota_n = jax.lax.broadcasted_iota(jnp.int32, (tb, _N), 1)
    idx = jnp.min(jnp.where(score == rowmax, iota_n, _N),
                  axis=1, keepdims=True)
    ss = jax.nn.sigmoid(rowmax)

    nh = _N * _H
    vals = jnp.concatenate(
        [u_all, jnp.ones((tb, _N), dtype=f32)], axis=1)
    lbl = jax.lax.broadcasted_iota(jnp.int32, (tb, nh + _N), 1)
    lbl = jnp.where(lbl < nh, lbl // _H, lbl - nh)
    masked = (jnp.where(lbl == idx, vals, 0.0) * ss).astype(bf16)
    upd_ss = jax.lax.dot_general(
        masked, gcat_ref[...].astype(bf16), _DN_N,
        preferred_element_type=f32)

    out_ref[...] = (1.0 - ss) * x_lin + _ETA * upd_ss


def kernel(x, W_lin, b_lin, select_weight, select_mean_diff, imp_slope,
           imp_center, lora_update, lora_degen, bias_p, debias_p):
    B, T, D = x.shape
    N, _, S = select_weight.shape
    H = lora_update.shape[2]
    x2 = x.reshape(B * T, D)
    b2 = b_lin.reshape(1, D)
    slope = imp_slope.reshape(1, N)
    center = imp_center.reshape(1, N)

    wsel = jnp.transpose(select_weight, (1, 0, 2)).reshape(D, N * S)
    u2 = jnp.transpose(lora_update, (1, 0, 2)).reshape(D, N * H)
    wcat = jnp.concatenate([u2, wsel, select_mean_diff.T], axis=1)
    mw = jnp.einsum('nd,nds->ns', select_mean_diff,
                    select_weight).reshape(1, N * S)
    m2 = jnp.sum(select_mean_diff * select_mean_diff, axis=1).reshape(1, N)
    c = jnp.einsum('nd,ndh->nh', debias_p, lora_update)
    cb = jnp.einsum('nh,ndh->nd', c, lora_degen)
    g2 = jnp.transpose(lora_degen, (0, 2, 1)).reshape(N * H, D)
    gcat = jnp.concatenate([g2, bias_p - cb], axis=0)

    TB = 512
    grid = ((B * T) // TB,)
    const = lambda shape: pl.BlockSpec(shape, lambda i: (0, 0))
    out = pl.pallas_call(
        _glo_kernel,
        grid=grid,
        in_specs=[
            pl.BlockSpec((TB, D), lambda i: (i, 0)),
            const((D, D)),
            const((1, D)),
            const((D, _NK)),
            const((1, N * S)),
            const((1, N)),
            const((1, N)),
            const((1, N)),
            const((N * (H + 1), D)),
        ],
        out_specs=pl.BlockSpec((TB, D), lambda i: (i, 0)),
        out_shape=jax.ShapeDtypeStruct((B * T, D), jnp.float32),
        scratch_shapes=[
            pltpu.VMEM((D + _NK, D), jnp.bfloat16),
            pltpu.VMEM((1, _NK), jnp.float32),
        ],
        compiler_params=pltpu.CompilerParams(
            dimension_semantics=("arbitrary",)),
    )(x2, W_lin, b2, wcat, mw, m2, slope, center, gcat)
    return out.reshape(B, T, D)
